# baseline (speedup 1.0000x reference)
import functools

import jax
import jax.numpy as jnp
from jax import lax
from jax.experimental import pallas as pl
from jax.experimental.pallas import tpu as pltpu
from jax.experimental.pallas import tpu_sc as plsc

_NC = 2
_NS = 16
_NW = _NC * _NS
_LANES = 16


def _mlp_body(tab_ref, w1_ref, b1_ref, w2_ref, b2_ref, out_ref):
    h = jnp.dot(tab_ref[...], w1_ref[...], preferred_element_type=jnp.float32)
    h = jnp.maximum(h + b1_ref[...], 0.0)
    o = jnp.dot(h, w2_ref[...], preferred_element_type=jnp.float32)
    out_ref[...] = o + b2_ref[...]


def _mlp_table(table, W1, b1, W2, b2):
    V, D = table.shape
    return pl.pallas_call(
        _mlp_body,
        out_shape=jax.ShapeDtypeStruct((V, D), jnp.float32),
    )(table, W1, b1.reshape(1, D), W2, b2.reshape(1, D))


@functools.lru_cache(maxsize=None)
def _make_gather_t(V, D, B, H):
    assert B % (_NW * 128) == 0 and D % 8 == 0
    b_per_w = B // _NW
    n_kb = b_per_w // 128
    n_blocks = H * n_kb
    assert n_blocks % 2 == 0
    mesh = plsc.VectorSubcoreMesh(
        core_axis_name="c", subcore_axis_name="s",
        num_cores=_NC, num_subcores=_NS,
    )

    @functools.partial(
        pl.kernel,
        mesh=mesh,
        out_type=jax.ShapeDtypeStruct((H, D // 8, B // 128, 8, 128),
                                      jnp.float32),
        compiler_params=pltpu.CompilerParams(use_tc_tiling_on_sc=False,
                                             needs_layout_passes=False),
        scratch_types=[
            pltpu.VMEM((V * D,), jnp.float32),
            pltpu.VMEM((H * b_per_w,), jnp.int32),
            pltpu.VMEM((D // 8, 8, 128), jnp.float32),
            pltpu.VMEM((D // 8, 8, 128), jnp.float32),
            pltpu.SemaphoreType.DMA,
            pltpu.SemaphoreType.DMA,
            pltpu.SemaphoreType.DMA,
        ],
    )
    def gather(tab_hbm, idx_hbm, out_hbm, tab_v, idx_v, st0, st1,
               s_idx, so0, so1):
        wid = lax.axis_index("s") * _NC + lax.axis_index("c")
        b0 = wid * b_per_w
        st = (st0, st1)
        so = (so0, so1)

        for h in range(H):
            pltpu.async_copy(
                idx_hbm.at[pl.ds(h * B + b0, b_per_w)],
                idx_v.at[pl.ds(h * b_per_w, b_per_w)], s_idx)
        pltpu.sync_copy(tab_hbm, tab_v)
        for h in range(H):
            pltpu.make_async_copy(
                idx_hbm.at[pl.ds(h * B + b0, b_per_w)],
                idx_v.at[pl.ds(h * b_per_w, b_per_w)], s_idx).wait()

        kb0 = b0 // 128

        def block(t, par):
            h = t // n_kb
            kb = lax.rem(t, n_kb)
            dst = out_hbm.at[h, pl.ds(0, D // 8), kb0 + kb]
            st_ref = st[par]

            @pl.when(t >= 2)
            def _():
                pltpu.make_async_copy(st_ref, dst, so[par]).wait()

            ib = h * b_per_w + kb * 128
            for g in range(128 // _LANES):
                iv = idx_v[pl.ds(ib + g * _LANES, _LANES)]
                addr = iv * D
                for d in range(D):
                    v = plsc.load_gather(tab_v, [addr + d])
                    st_ref[d // 8, d % 8, pl.ds(g * _LANES, _LANES)] = v
            pltpu.async_copy(st_ref, dst, so[par])

        def body(t2, carry):
            block(t2 * 2, 0)
            block(t2 * 2 + 1, 1)
            return carry

        lax.fori_loop(0, n_blocks // 2, body, 0)

        last = out_hbm.at[H - 1, pl.ds(0, D // 8), kb0]
        pltpu.make_async_copy(st0, last, so0).wait()
        pltpu.make_async_copy(st1, last, so1).wait()

    return gather


def kernel(emotion_ids, table, W1, b1, W2, b2):
    Bb, H = emotion_ids.shape
    V, D = table.shape
    mlp_tab = _mlp_table(table, W1, b1, W2, b2)
    tab_flat = mlp_tab.reshape(-1)
    idx_t = emotion_ids.T.reshape(-1).astype(jnp.int32)
    out5 = _make_gather_t(V, D, Bb, H)(tab_flat, idx_t)
    return jnp.transpose(out5, (2, 4, 0, 1, 3)).reshape(Bb, H, D)


# speedup vs baseline: 1.6426x; 1.6426x over previous
import functools

import jax
import jax.numpy as jnp
from jax import lax
from jax.experimental import pallas as pl
from jax.experimental.pallas import tpu as pltpu
from jax.experimental.pallas import tpu_sc as plsc

_NC = 2
_NS = 16
_NW = _NC * _NS
_LANES = 16


def _mlp_body(tab_ref, w1_ref, b1_ref, w2_ref, b2_ref, out_ref):
    h = jnp.dot(tab_ref[...], w1_ref[...], preferred_element_type=jnp.float32)
    h = jnp.maximum(h + b1_ref[...], 0.0)
    o = jnp.dot(h, w2_ref[...], preferred_element_type=jnp.float32)
    out_ref[...] = o + b2_ref[...]


def _mlp_table(table, W1, b1, W2, b2):
    V, D = table.shape
    return pl.pallas_call(
        _mlp_body,
        out_shape=jax.ShapeDtypeStruct((V, D), jnp.float32),
    )(table, W1, b1.reshape(1, D), W2, b2.reshape(1, D))


@functools.lru_cache(maxsize=None)
def _make_gather_t(V, D, B, H):
    assert B % (_NW * 128) == 0 and D % 8 == 0
    b_per_w = B // _NW
    n_kb = b_per_w // 128
    n_blocks = H * n_kb
    assert n_blocks % 2 == 0
    mesh = plsc.VectorSubcoreMesh(
        core_axis_name="c", subcore_axis_name="s",
        num_cores=_NC, num_subcores=_NS,
    )

    @functools.partial(
        pl.kernel,
        mesh=mesh,
        out_type=jax.ShapeDtypeStruct((H, D // 8, B // 128, 8, 128),
                                      jnp.float32),
        compiler_params=pltpu.CompilerParams(use_tc_tiling_on_sc=False,
                                             needs_layout_passes=False),
        scratch_types=[
            pltpu.VMEM((V * D,), jnp.float32),
            pltpu.VMEM((H * b_per_w,), jnp.int32),
            pltpu.VMEM((D // 8, 8, 128), jnp.float32),
            pltpu.VMEM((D // 8, 8, 128), jnp.float32),
            pltpu.SemaphoreType.DMA,
            pltpu.SemaphoreType.DMA,
            pltpu.SemaphoreType.DMA,
        ],
    )
    def gather(tab_hbm, idx_hbm, out_hbm, tab_v, idx_v, st0, st1,
               s_idx, so0, so1):
        wid = lax.axis_index("s") * _NC + lax.axis_index("c")
        b0 = wid * b_per_w
        st = (st0, st1)
        so = (so0, so1)

        for h in range(H):
            pltpu.async_copy(
                idx_hbm.at[pl.ds(h * B + b0, b_per_w)],
                idx_v.at[pl.ds(h * b_per_w, b_per_w)], s_idx)
        pltpu.sync_copy(tab_hbm, tab_v)
        for h in range(H):
            pltpu.make_async_copy(
                idx_hbm.at[pl.ds(h * B + b0, b_per_w)],
                idx_v.at[pl.ds(h * b_per_w, b_per_w)], s_idx).wait()

        kb0 = b0 // 128

        def block(t, par):
            h = t // n_kb
            kb = lax.rem(t, n_kb)
            dst = out_hbm.at[h, pl.ds(0, D // 8), kb0 + kb]
            st_ref = st[par]

            @pl.when(t >= 2)
            def _():
                pltpu.make_async_copy(st_ref, dst, so[par]).wait()

            ib = h * b_per_w + kb * 128
            for g in range(128 // _LANES):
                iv = idx_v[pl.ds(ib + g * _LANES, _LANES)]
                addr = iv * D
                for d0 in range(0, D, 8):
                    vs = [plsc.load_gather(tab_v, [addr + (d0 + j)])
                          for j in range(8)]
                    for j in range(8):
                        st_ref[(d0 + j) // 8, (d0 + j) % 8,
                               pl.ds(g * _LANES, _LANES)] = vs[j]
            pltpu.async_copy(st_ref, dst, so[par])

        def body(t2, carry):
            block(t2 * 2, 0)
            block(t2 * 2 + 1, 1)
            return carry

        lax.fori_loop(0, n_blocks // 2, body, 0)

        last = out_hbm.at[H - 1, pl.ds(0, D // 8), kb0]
        pltpu.make_async_copy(st0, last, so0).wait()
        pltpu.make_async_copy(st1, last, so1).wait()

    return gather


def kernel(emotion_ids, table, W1, b1, W2, b2):
    Bb, H = emotion_ids.shape
    V, D = table.shape
    mlp_tab = _mlp_table(table, W1, b1, W2, b2)
    tab_flat = mlp_tab.reshape(-1)
    idx_t = emotion_ids.T.reshape(-1).astype(jnp.int32)
    out5 = _make_gather_t(V, D, Bb, H)(tab_flat, idx_t)
    return jnp.transpose(out5, (2, 4, 0, 1, 3)).reshape(Bb, H, D)


# speedup vs baseline: 3.5135x; 2.1389x over previous
import functools

import jax
import jax.numpy as jnp
from jax import lax
from jax.experimental import pallas as pl
from jax.experimental.pallas import tpu as pltpu
from jax.experimental.pallas import tpu_sc as plsc

_NC = 2
_NS = 16
_NW = _NC * _NS
_LANES = 16


def _mlp_body(tab_ref, w1_ref, b1_ref, w2_ref, b2_ref, out_ref):
    h = jnp.dot(tab_ref[...], w1_ref[...], preferred_element_type=jnp.float32)
    h = jnp.maximum(h + b1_ref[...], 0.0)
    o = jnp.dot(h, w2_ref[...], preferred_element_type=jnp.float32)
    out_ref[...] = o + b2_ref[...]


def _mlp_table(table, W1, b1, W2, b2):
    V, D = table.shape
    return pl.pallas_call(
        _mlp_body,
        out_shape=jax.ShapeDtypeStruct((V, D), jnp.float32),
    )(table, W1, b1.reshape(1, D), W2, b2.reshape(1, D))


@functools.lru_cache(maxsize=None)
def _make_gather_t(V, D, B, H):
    assert B % (_NW * 128) == 0 and D % 8 == 0
    b_per_w = B // _NW
    n_kb = b_per_w // 128
    n_blocks = H * n_kb
    assert n_blocks % 2 == 0
    mesh = plsc.VectorSubcoreMesh(
        core_axis_name="c", subcore_axis_name="s",
        num_cores=_NC, num_subcores=_NS,
    )

    @functools.partial(
        pl.kernel,
        mesh=mesh,
        out_type=jax.ShapeDtypeStruct((H, D // 8, B // 128, 8, 128),
                                      jnp.float32),
        compiler_params=pltpu.CompilerParams(use_tc_tiling_on_sc=False,
                                             needs_layout_passes=False),
        scratch_types=[
            pltpu.VMEM((V * (D + 1),), jnp.float32),
            pltpu.VMEM((H * b_per_w,), jnp.int32),
            pltpu.VMEM((D // 8, 8, 128), jnp.float32),
            pltpu.VMEM((D // 8, 8, 128), jnp.float32),
            pltpu.SemaphoreType.DMA,
            pltpu.SemaphoreType.DMA,
            pltpu.SemaphoreType.DMA,
        ],
    )
    def gather(tab_hbm, idx_hbm, out_hbm, tab_v, idx_v, st0, st1,
               s_idx, so0, so1):
        wid = lax.axis_index("s") * _NC + lax.axis_index("c")
        b0 = wid * b_per_w
        st = (st0, st1)
        so = (so0, so1)

        for h in range(H):
            pltpu.async_copy(
                idx_hbm.at[pl.ds(h * B + b0, b_per_w)],
                idx_v.at[pl.ds(h * b_per_w, b_per_w)], s_idx)
        pltpu.sync_copy(tab_hbm, tab_v)
        for h in range(H):
            pltpu.make_async_copy(
                idx_hbm.at[pl.ds(h * B + b0, b_per_w)],
                idx_v.at[pl.ds(h * b_per_w, b_per_w)], s_idx).wait()

        kb0 = b0 // 128

        def block(t, par):
            h = t // n_kb
            kb = lax.rem(t, n_kb)
            dst = out_hbm.at[h, pl.ds(0, D // 8), kb0 + kb]
            st_ref = st[par]

            @pl.when(t >= 2)
            def _():
                pltpu.make_async_copy(st_ref, dst, so[par]).wait()

            ib = h * b_per_w + kb * 128
            for g in range(128 // _LANES):
                iv = idx_v[pl.ds(ib + g * _LANES, _LANES)]
                addr = iv * (D + 1)
                for d0 in range(0, D, 8):
                    vs = [plsc.load_gather(tab_v, [addr + (d0 + j)])
                          for j in range(8)]
                    for j in range(8):
                        st_ref[(d0 + j) // 8, (d0 + j) % 8,
                               pl.ds(g * _LANES, _LANES)] = vs[j]
            pltpu.async_copy(st_ref, dst, so[par])

        def body(t2, carry):
            block(t2 * 2, 0)
            block(t2 * 2 + 1, 1)
            return carry

        lax.fori_loop(0, n_blocks // 2, body, 0)

        last = out_hbm.at[H - 1, pl.ds(0, D // 8), kb0]
        pltpu.make_async_copy(st0, last, so0).wait()
        pltpu.make_async_copy(st1, last, so1).wait()

    return gather


def kernel(emotion_ids, table, W1, b1, W2, b2):
    Bb, H = emotion_ids.shape
    V, D = table.shape
    mlp_tab = _mlp_table(table, W1, b1, W2, b2)
    tab_flat = jnp.pad(mlp_tab, ((0, 0), (0, 1))).reshape(-1)
    idx_t = emotion_ids.T.reshape(-1).astype(jnp.int32)
    out5 = _make_gather_t(V, D, Bb, H)(tab_flat, idx_t)
    return jnp.transpose(out5, (2, 4, 0, 1, 3)).reshape(Bb, H, D)
